# Initial kernel scaffold; baseline (speedup 1.0000x reference)
import jax
import jax.numpy as jnp
from jax.experimental import pallas as pl


def kernel(peptide_tokens, decoration, charge, a_emb, phos_emb, charge_emb, pos_emb, ln_gamma, ln_beta):
    raise NotImplementedError("write your pallas kernel here")


# speedup vs baseline: 3.5787x; 3.5787x over previous
import jax
import jax.numpy as jnp
from jax.experimental import pallas as pl

N, L, D = 16384, 50, 128
EPS = 1e-12
R = 3200


def _body(tok_ref, dec_ref, chg_ref, tbl_ref, g_ref, b_ref, out_ref):
    li = jax.lax.broadcasted_iota(jnp.int32, (R, D), 1)
    ri = jax.lax.broadcasted_iota(jnp.int32, (R, D), 0)
    l_idx = ri % L
    oh = ((tok_ref[...] == li).astype(jnp.float32)
          + ((dec_ref[...] + 30) == li).astype(jnp.float32)
          + ((chg_ref[...] + 40) == li).astype(jnp.float32)
          + ((l_idx + 50) == li).astype(jnp.float32))
    e = jnp.dot(oh, tbl_ref[...], preferred_element_type=jnp.float32)
    m = jnp.mean(e, axis=1, keepdims=True)
    d = e - m
    v = jnp.mean(d * d, axis=1, keepdims=True)
    r = jax.lax.rsqrt(v + EPS)
    out_ref[...] = d * r * g_ref[...] + b_ref[...]


def kernel(peptide_tokens, decoration, charge, a_emb, phos_emb, charge_emb,
           pos_emb, ln_gamma, ln_beta):
    total = N * L
    tok = peptide_tokens.reshape(total, 1).astype(jnp.int32)
    dec = decoration.reshape(total, 1).astype(jnp.int32)
    chg = jnp.repeat(charge.astype(jnp.int32), L).reshape(total, 1)
    tbl = jnp.concatenate(
        [a_emb, phos_emb, charge_emb, pos_emb[:L],
         jnp.zeros((D - 100, D), jnp.float32)], axis=0)
    g = ln_gamma.reshape(1, D)
    b = ln_beta.reshape(1, D)

    grid = (total // R,)
    out = pl.pallas_call(
        _body,
        grid=grid,
        in_specs=[
            pl.BlockSpec((R, 1), lambda i: (i, 0)),
            pl.BlockSpec((R, 1), lambda i: (i, 0)),
            pl.BlockSpec((R, 1), lambda i: (i, 0)),
            pl.BlockSpec((D, D), lambda i: (0, 0)),
            pl.BlockSpec((1, D), lambda i: (0, 0)),
            pl.BlockSpec((1, D), lambda i: (0, 0)),
        ],
        out_specs=pl.BlockSpec((R, D), lambda i: (i, 0)),
        out_shape=jax.ShapeDtypeStruct((total, D), jnp.float32),
    )(tok, dec, chg, tbl, g, b)
    return out.reshape(N, L, D)


# speedup vs baseline: 4.4152x; 1.2337x over previous
import jax
import jax.numpy as jnp
from jax import lax
from jax.experimental import pallas as pl

N, L, D = 16384, 50, 128
EPS = 1e-12
R = 3200
CH = R // 128


def _body(tok_ref, dec_ref, chg_ref, tbl_ref, pos_ref, g_ref, b_ref, out_ref):
    vi = lax.broadcasted_iota(jnp.int32, (D, 128), 0)
    g = g_ref[...]
    b = b_ref[...]
    for c in range(CH):
        tok = jnp.broadcast_to(tok_ref[0, c:c + 1, :], (D, 128))
        dec = jnp.broadcast_to(dec_ref[0, c:c + 1, :], (D, 128))
        chg = jnp.broadcast_to(chg_ref[0, c:c + 1, :], (D, 128))
        oht = ((tok == vi).astype(jnp.float32)
               + (dec == vi).astype(jnp.float32)
               + (chg == vi).astype(jnp.float32))
        e = lax.dot_general(oht, tbl_ref[...], (((0,), (0,)), ((), ())),
                            preferred_element_type=jnp.float32)
        e = e + pos_ref[c * 128:(c + 1) * 128, :]
        m = jnp.mean(e, axis=1, keepdims=True)
        d = e - m
        v = jnp.mean(d * d, axis=1, keepdims=True)
        r = lax.rsqrt(v + EPS)
        out_ref[c * 128:(c + 1) * 128, :] = d * r * g + b


def kernel(peptide_tokens, decoration, charge, a_emb, phos_emb, charge_emb,
           pos_emb, ln_gamma, ln_beta):
    total = N * L
    nb = total // R
    tok = peptide_tokens.reshape(nb, CH, 128).astype(jnp.int32)
    dec = decoration.reshape(nb, CH, 128).astype(jnp.int32) + 30
    chg = jnp.repeat(charge.astype(jnp.int32), L).reshape(nb, CH, 128) + 40
    tbl = jnp.concatenate(
        [a_emb, phos_emb, charge_emb, jnp.zeros((D - 50, D), jnp.float32)],
        axis=0)
    pos_tile = jnp.tile(pos_emb[:L], (R // L, 1))
    g = ln_gamma.reshape(1, D)
    b = ln_beta.reshape(1, D)

    grid = (total // R,)
    out = pl.pallas_call(
        _body,
        grid=grid,
        in_specs=[
            pl.BlockSpec((1, CH, 128), lambda i: (i, 0, 0)),
            pl.BlockSpec((1, CH, 128), lambda i: (i, 0, 0)),
            pl.BlockSpec((1, CH, 128), lambda i: (i, 0, 0)),
            pl.BlockSpec((D, D), lambda i: (0, 0)),
            pl.BlockSpec((R, D), lambda i: (0, 0)),
            pl.BlockSpec((1, D), lambda i: (0, 0)),
            pl.BlockSpec((1, D), lambda i: (0, 0)),
        ],
        out_specs=pl.BlockSpec((R, D), lambda i: (i, 0)),
        out_shape=jax.ShapeDtypeStruct((total, D), jnp.float32),
    )(tok, dec, chg, tbl, pos_tile, g, b)
    return out.reshape(N, L, D)


# speedup vs baseline: 6.9119x; 1.5655x over previous
import jax
import jax.numpy as jnp
from jax import lax
from jax.experimental import pallas as pl

N, L, D = 16384, 50, 128
EPS = 1e-12
R = 3200
CH = R // 128


def _body(tok_ref, dec_ref, chg_ref, tbl_ref, pos_ref, g_ref, b_ref, out_ref):
    vi = lax.broadcasted_iota(jnp.int32, (D, 128), 0)
    g = g_ref[...]
    b = b_ref[...]
    ones = jnp.full((D, D), 1.0 / D, dtype=jnp.float32)
    for c in range(CH):
        tok = jnp.broadcast_to(tok_ref[0, c:c + 1, :], (D, 128))
        dec = jnp.broadcast_to(dec_ref[0, c:c + 1, :], (D, 128))
        chg = jnp.broadcast_to(chg_ref[0, c:c + 1, :], (D, 128))
        oht = ((tok == vi).astype(jnp.float32)
               + (dec == vi).astype(jnp.float32)
               + (chg == vi).astype(jnp.float32))
        d = lax.dot_general(oht, tbl_ref[...], (((0,), (0,)), ((), ())),
                            preferred_element_type=jnp.float32)
        d = d + pos_ref[c * 128:(c + 1) * 128, :]
        v = jnp.dot(d * d, ones, preferred_element_type=jnp.float32)
        r = lax.rsqrt(v + EPS)
        out_ref[c * 128:(c + 1) * 128, :] = d * r * g + b


def kernel(peptide_tokens, decoration, charge, a_emb, phos_emb, charge_emb,
           pos_emb, ln_gamma, ln_beta):
    total = N * L
    nb = total // R
    tok = peptide_tokens.reshape(nb, CH, 128).astype(jnp.int32)
    dec = decoration.reshape(nb, CH, 128).astype(jnp.int32) + 30
    chg = jnp.repeat(charge.astype(jnp.int32), L).reshape(nb, CH, 128) + 40
    tbl = jnp.concatenate(
        [a_emb, phos_emb, charge_emb, jnp.zeros((D - 50, D), jnp.float32)],
        axis=0)
    pos_tile = jnp.tile(pos_emb[:L], (R // L, 1))
    cen = jnp.eye(D, dtype=jnp.float32) - 1.0 / D
    tbl = tbl @ cen
    pos_tile = pos_tile @ cen
    g = ln_gamma.reshape(1, D)
    b = ln_beta.reshape(1, D)

    grid = (total // R,)
    out = pl.pallas_call(
        _body,
        grid=grid,
        in_specs=[
            pl.BlockSpec((1, CH, 128), lambda i: (i, 0, 0)),
            pl.BlockSpec((1, CH, 128), lambda i: (i, 0, 0)),
            pl.BlockSpec((1, CH, 128), lambda i: (i, 0, 0)),
            pl.BlockSpec((D, D), lambda i: (0, 0)),
            pl.BlockSpec((R, D), lambda i: (0, 0)),
            pl.BlockSpec((1, D), lambda i: (0, 0)),
            pl.BlockSpec((1, D), lambda i: (0, 0)),
        ],
        out_specs=pl.BlockSpec((R, D), lambda i: (i, 0)),
        out_shape=jax.ShapeDtypeStruct((total, D), jnp.float32),
    )(tok, dec, chg, tbl, pos_tile, g, b)
    return out.reshape(N, L, D)


# speedup vs baseline: 12.4263x; 1.7978x over previous
import jax
import jax.numpy as jnp
from jax import lax
from jax.experimental import pallas as pl

N, L, D = 16384, 50, 128
EPS = 1e-12
R = 3200
CH = R // 128


def _body(tok_ref, dec_ref, chg_ref, tbl_ref, pos_ref, g_ref, b_ref, out_ref):
    vi = lax.broadcasted_iota(jnp.int32, (D, 128), 0)
    g = g_ref[...]
    b = b_ref[...]
    ones = jnp.full((D, D), 1.0 / D, dtype=jnp.float32)
    for c in range(CH):
        tok = jnp.broadcast_to(tok_ref[0, c:c + 1, :], (D, 128))
        dec = jnp.broadcast_to(dec_ref[0, c:c + 1, :], (D, 128))
        chg = jnp.broadcast_to(chg_ref[0, c:c + 1, :], (D, 128))
        oht = ((tok == vi).astype(jnp.float32)
               + (dec == vi).astype(jnp.float32)
               + (chg == vi).astype(jnp.float32))
        d = lax.dot_general(oht, tbl_ref[...], (((0,), (0,)), ((), ())),
                            preferred_element_type=jnp.float32)
        d = d + pos_ref[c * 128:(c + 1) * 128, :]
        v = jnp.dot(d * d, ones, preferred_element_type=jnp.float32)
        r = lax.rsqrt(v + EPS)
        o = d * r * g + b
        j = 0
        while j < 128:
            flat = c * 128 + j
            n_p, l0 = flat // L, flat % L
            ln = min(L - l0, 128 - j)
            out_ref[n_p, l0:l0 + ln, :] = o[j:j + ln, :]
            j += ln


def kernel(peptide_tokens, decoration, charge, a_emb, phos_emb, charge_emb,
           pos_emb, ln_gamma, ln_beta):
    total = N * L
    nb = total // R
    tok = peptide_tokens.reshape(nb, CH, 128).astype(jnp.int32)
    dec = decoration.reshape(nb, CH, 128).astype(jnp.int32) + 30
    chg = jnp.repeat(charge.astype(jnp.int32), L).reshape(nb, CH, 128) + 40
    tbl = jnp.concatenate(
        [a_emb, phos_emb, charge_emb, jnp.zeros((D - 50, D), jnp.float32)],
        axis=0)
    pos_tile = jnp.tile(pos_emb[:L], (R // L, 1))
    cen = jnp.eye(D, dtype=jnp.float32) - 1.0 / D
    tbl = tbl @ cen
    pos_tile = pos_tile @ cen
    g = ln_gamma.reshape(1, D)
    b = ln_beta.reshape(1, D)

    grid = (total // R,)
    out = pl.pallas_call(
        _body,
        grid=grid,
        in_specs=[
            pl.BlockSpec((1, CH, 128), lambda i: (i, 0, 0)),
            pl.BlockSpec((1, CH, 128), lambda i: (i, 0, 0)),
            pl.BlockSpec((1, CH, 128), lambda i: (i, 0, 0)),
            pl.BlockSpec((D, D), lambda i: (0, 0)),
            pl.BlockSpec((R, D), lambda i: (0, 0)),
            pl.BlockSpec((1, D), lambda i: (0, 0)),
            pl.BlockSpec((1, D), lambda i: (0, 0)),
        ],
        out_specs=pl.BlockSpec((R // L, L, D), lambda i: (i, 0, 0)),
        out_shape=jax.ShapeDtypeStruct((N, L, D), jnp.float32),
    )(tok, dec, chg, tbl, pos_tile, g, b)
    return out


# speedup vs baseline: 13.3150x; 1.0715x over previous
import jax
import jax.numpy as jnp
from jax import lax
from jax.experimental import pallas as pl

N, L, D = 16384, 50, 128
EPS = 1e-12
R = 3200
CH = R // 128


def _body(pk_ref, tbl_ref, pos_ref, g_ref, b_ref, out_ref):
    vi = lax.broadcasted_iota(jnp.int32, (D, 128), 0)
    g = g_ref[...]
    b = b_ref[...]
    ones = jnp.full((D, D), 1.0 / D, dtype=jnp.float32)
    for c in range(CH):
        pk = pk_ref[0, c:c + 1, :]
        tok = jnp.broadcast_to(pk & 127, (D, 128))
        dec = jnp.broadcast_to((pk >> 7) & 127, (D, 128))
        chg = jnp.broadcast_to(pk >> 14, (D, 128))
        oht = ((tok == vi).astype(jnp.float32)
               + (dec == vi).astype(jnp.float32)
               + (chg == vi).astype(jnp.float32))
        d = lax.dot_general(oht, tbl_ref[...], (((0,), (0,)), ((), ())),
                            preferred_element_type=jnp.float32)
        d = d + pos_ref[c * 128:(c + 1) * 128, :]
        v = jnp.dot(d * d, ones, preferred_element_type=jnp.float32)
        r = lax.rsqrt(v + EPS)
        o = d * r * g + b
        j = 0
        while j < 128:
            flat = c * 128 + j
            n_p, l0 = flat // L, flat % L
            ln = min(L - l0, 128 - j)
            out_ref[n_p, l0:l0 + ln, :] = o[j:j + ln, :]
            j += ln


def kernel(peptide_tokens, decoration, charge, a_emb, phos_emb, charge_emb,
           pos_emb, ln_gamma, ln_beta):
    total = N * L
    nb = total // R
    packed = (peptide_tokens.astype(jnp.int32)
              + ((decoration.astype(jnp.int32) + 30) << 7)
              + ((charge.astype(jnp.int32)[:, None] + 40) << 14))
    packed = packed.reshape(nb, CH, 128)
    tbl = jnp.concatenate(
        [a_emb, phos_emb, charge_emb, jnp.zeros((D - 50, D), jnp.float32)],
        axis=0)
    pos_tile = jnp.tile(pos_emb[:L], (R // L, 1))
    cen = jnp.eye(D, dtype=jnp.float32) - 1.0 / D
    tbl = tbl @ cen
    pos_tile = pos_tile @ cen
    g = ln_gamma.reshape(1, D)
    b = ln_beta.reshape(1, D)

    grid = (total // R,)
    out = pl.pallas_call(
        _body,
        grid=grid,
        in_specs=[
            pl.BlockSpec((1, CH, 128), lambda i: (i, 0, 0)),
            pl.BlockSpec((D, D), lambda i: (0, 0)),
            pl.BlockSpec((R, D), lambda i: (0, 0)),
            pl.BlockSpec((1, D), lambda i: (0, 0)),
            pl.BlockSpec((1, D), lambda i: (0, 0)),
        ],
        out_specs=pl.BlockSpec((R // L, L, D), lambda i: (i, 0, 0)),
        out_shape=jax.ShapeDtypeStruct((N, L, D), jnp.float32),
    )(packed, tbl, pos_tile, g, b)
    return out


# speedup vs baseline: 30.5949x; 2.2978x over previous
import jax
import jax.numpy as jnp
from jax import lax
from jax.experimental import pallas as pl

N, L, D = 16384, 50, 128
EPS = 1e-12
NB = 128


def _body(pk_ref, tbl_ref, pos_ref, g_ref, b_ref, out_ref):
    vi = lax.broadcasted_iota(jnp.int32, (D, NB), 0)
    g = g_ref[...]
    b = b_ref[...]
    ones = jnp.full((D, D), 1.0 / D, dtype=jnp.float32)
    for l in range(L):
        pk = pk_ref[l, 0, 0:1, :]
        tok = jnp.broadcast_to(pk & 127, (D, NB))
        dec = jnp.broadcast_to((pk >> 7) & 127, (D, NB))
        chg = jnp.broadcast_to(pk >> 14, (D, NB))
        oht = ((tok == vi).astype(jnp.float32)
               + (dec == vi).astype(jnp.float32)
               + (chg == vi).astype(jnp.float32))
        d = lax.dot_general(oht, tbl_ref[...], (((0,), (0,)), ((), ())),
                            preferred_element_type=jnp.float32)
        d = d + jnp.broadcast_to(pos_ref[l:l + 1, :], (NB, D))
        v = jnp.dot(d * d, ones, preferred_element_type=jnp.float32)
        r = lax.rsqrt(v + EPS)
        out_ref[l, :, :] = d * r * g + b


def kernel(peptide_tokens, decoration, charge, a_emb, phos_emb, charge_emb,
           pos_emb, ln_gamma, ln_beta):
    nc = N // NB
    packed = (peptide_tokens.astype(jnp.int32)
              + ((decoration.astype(jnp.int32) + 30) << 7)
              + ((charge.astype(jnp.int32)[:, None] + 40) << 14))
    pkt = packed.T.reshape(L, nc, 1, NB)
    tbl = jnp.concatenate(
        [a_emb, phos_emb, charge_emb, jnp.zeros((D - 50, D), jnp.float32)],
        axis=0)
    cen = jnp.eye(D, dtype=jnp.float32) - 1.0 / D
    tbl = tbl @ cen
    pos = pos_emb[:L] @ cen
    g = ln_gamma.reshape(1, D)
    b = ln_beta.reshape(1, D)

    out = pl.pallas_call(
        _body,
        grid=(nc,),
        in_specs=[
            pl.BlockSpec((L, 1, 1, NB), lambda i: (0, i, 0, 0)),
            pl.BlockSpec((D, D), lambda i: (0, 0)),
            pl.BlockSpec((L, D), lambda i: (0, 0)),
            pl.BlockSpec((1, D), lambda i: (0, 0)),
            pl.BlockSpec((1, D), lambda i: (0, 0)),
        ],
        out_specs=pl.BlockSpec((L, NB, D), lambda i: (0, i, 0)),
        out_shape=jax.ShapeDtypeStruct((L, N, D), jnp.float32),
    )(pkt, tbl, pos, g, b)
    return jnp.transpose(out, (1, 0, 2))
